# baseline (speedup 1.0000x reference)
import functools

import jax
import jax.numpy as jnp
from jax import lax
from jax.experimental import pallas as pl
from jax.experimental.pallas import tpu as pltpu
from jax.experimental.pallas import tpu_sc as plsc

NS = [10000, 2500, 640, 160, 40]
ES = [160000, 40000, 10000, 2500, 600]
G = 8
K = 5
NCORE = 2
NSUB = 16
NW = NCORE * NSUB
LANES = 16


def _ceil_to(a, m):
    return -(-a // m) * m


def _corners(p0, p1, p2):
    q0, q1, q2 = p0 * 4.0, p1 * 4.0, p2 * 4.0
    lo0 = jnp.minimum(jnp.maximum(q0.astype(jnp.int32), 0), 3)
    lo1 = jnp.minimum(jnp.maximum(q1.astype(jnp.int32), 0), 3)
    lo2 = jnp.minimum(jnp.maximum(q2.astype(jnp.int32), 0), 3)
    fr0 = q0 - lo0.astype(jnp.float32)
    fr1 = q1 - lo1.astype(jnp.float32)
    fr2 = q2 - lo2.astype(jnp.float32)
    out = []
    for bits in range(8):
        b0, b1, b2 = (bits >> 2) & 1, (bits >> 1) & 1, bits & 1
        bin_ = (lo0 + b0) * 25 + (lo1 + b1) * 5 + (lo2 + b2)
        c0 = fr0 if b0 else 1.0 - fr0
        c1 = fr1 if b1 else 1.0 - fr1
        c2 = fr2 if b2 else 1.0 - fr2
        out.append((bin_, c0 * c1 * c2))
    return out


def _narrow_sc(Np, Ep, C):
    Wn = C + 16
    Epw = Ep // NW
    groups = Epw // LANES
    r = Np // NSUB
    mesh = plsc.VectorSubcoreMesh(core_axis_name="c", subcore_axis_name="s")

    def body(src_hbm, dst_hbm, p0_hbm, p1_hbm, p2_hbm, x_hbm, t_hbm, out_hbm,
             srcb, dstb, p0b, p1b, p2b, xb, tb, binb, cfb, msgb, zb,
             accum):
        cid = lax.axis_index("c")
        sid = lax.axis_index("s")
        wid = cid * NSUB + sid
        row0 = sid * r
        pltpu.sync_copy(src_hbm.at[pl.ds(wid * Epw, Epw)], srcb)
        pltpu.sync_copy(dst_hbm.at[pl.ds(wid * Epw, Epw)], dstb)
        pltpu.sync_copy(p0_hbm.at[pl.ds(wid * Epw, Epw)], p0b)
        pltpu.sync_copy(p1_hbm.at[pl.ds(wid * Epw, Epw)], p1b)
        pltpu.sync_copy(p2_hbm.at[pl.ds(wid * Epw, Epw)], p2b)
        pltpu.sync_copy(x_hbm, xb)
        pltpu.sync_copy(t_hbm, tb)

        zeros16 = jnp.zeros((16,), jnp.float32)
        iota = lax.iota(jnp.int32, 16)
        onehot = jnp.where(iota == 0, 1.0, 0.0).astype(jnp.float32)
        for i in range(LANES):
            for j in range(C // 16):
                msgb[i, pl.ds(j * 16, 16)] = zeros16
            msgb[i, pl.ds(C, 16)] = onehot
            for j in range(Wn // 16):
                zb[i, pl.ds(j * 16, 16)] = zeros16

        def zinit(i, carry):
            pltpu.sync_copy(zb, accum.at[pl.ds(row0 + i * 16, 16)])
            return carry
        lax.fori_loop(0, r // 16, zinit, 0)
        plsc.subcore_barrier()

        def grp(g, carry):
            base = g * LANES
            sv = srcb[pl.ds(base, 16)]
            dv = dstb[pl.ds(base, 16)]
            pa = p0b[pl.ds(base, 16)]
            pb_ = p1b[pl.ds(base, 16)]
            pc = p2b[pl.ds(base, 16)]
            xs = plsc.load_gather(xb, [sv])
            corners = _corners(pa, pb_, pc)
            for c8, (bin_, coef) in enumerate(corners):
                binb[pl.ds(c8 * 16, 16)] = bin_ * C
                cfb[pl.ds(c8 * 16, 16)] = coef * xs
            for e in range(LANES):
                accs = [None] * (C // 16)
                for c8 in range(8):
                    pos = jnp.full((16,), c8 * 16 + e, jnp.int32)
                    bs = plsc.load_gather(binb, [pos])
                    cs = plsc.load_gather(cfb, [pos])
                    for blk in range(C // 16):
                        v = plsc.load_gather(tb, [bs + (blk * 16) + iota])
                        accs[blk] = (cs * v if accs[blk] is None
                                     else accs[blk] + cs * v)
                for blk in range(C // 16):
                    msgb[e, pl.ds(blk * 16, 16)] = accs[blk]
            pltpu.sync_copy(msgb, accum.at[dv], add=True)
            return carry
        lax.fori_loop(0, groups, grp, 0)

        plsc.subcore_barrier()

        def cpout(i, carry):
            pltpu.sync_copy(accum.at[pl.ds(row0 + i * 16, 16)], zb)
            pltpu.sync_copy(zb, out_hbm.at[wid, pl.ds(i * 16, 16)])
            return carry
        lax.fori_loop(0, r // 16, cpout, 0)

    return pl.kernel(
        body, mesh=mesh,
        compiler_params=pltpu.CompilerParams(
            needs_layout_passes=False, use_tc_tiling_on_sc=False),
        out_type=jax.ShapeDtypeStruct((NW, r, Wn), jnp.float32),
        scratch_types=[
            pltpu.VMEM((Epw,), jnp.int32),
            pltpu.VMEM((Epw,), jnp.int32),
            pltpu.VMEM((Epw,), jnp.float32),
            pltpu.VMEM((Epw,), jnp.float32),
            pltpu.VMEM((Epw,), jnp.float32),
            pltpu.VMEM((Np,), jnp.float32),
            pltpu.VMEM((125 * C,), jnp.float32),
            pltpu.VMEM((8 * LANES,), jnp.int32),
            pltpu.VMEM((8 * LANES,), jnp.float32),
            pltpu.VMEM((LANES, Wn), jnp.float32),
            pltpu.VMEM((16, Wn), jnp.float32),
            pltpu.VMEM_SHARED((Np, Wn), jnp.float32),
        ],
    )


def _wide_sc(Np, Ep, C):
    SUB = 4
    CH = SUB * LANES
    Epw = Ep // NW
    nchunks = Epw // CH
    r = Np // NSUB
    mesh = plsc.VectorSubcoreMesh(core_axis_name="c", subcore_axis_name="s")

    def body(src_hbm, dst_hbm, p0_hbm, p1_hbm, p2_hbm, y_hbm, out_hbm,
             srcb, dstb, p0b, p1b, p2b, gis, cf, dvb, rowss, msgb, zb, sem,
             accum):
        cid = lax.axis_index("c")
        sid = lax.axis_index("s")
        wid = cid * NSUB + sid
        row0 = sid * r
        pltpu.sync_copy(src_hbm.at[pl.ds(wid * Epw, Epw)], srcb)
        pltpu.sync_copy(dst_hbm.at[pl.ds(wid * Epw, Epw)], dstb)
        pltpu.sync_copy(p0_hbm.at[pl.ds(wid * Epw, Epw)], p0b)
        pltpu.sync_copy(p1_hbm.at[pl.ds(wid * Epw, Epw)], p1b)
        pltpu.sync_copy(p2_hbm.at[pl.ds(wid * Epw, Epw)], p2b)

        zeros16 = jnp.zeros((16,), jnp.float32)
        iota = lax.iota(jnp.int32, 16)
        for i in range(16):
            for j in range(C // 16):
                zb[i, pl.ds(j * 16, 16)] = zeros16

        def zinit(i, carry):
            pltpu.sync_copy(zb, accum.at[pl.ds(row0 + i * 16, 16)])
            return carry
        lax.fori_loop(0, r // 16, zinit, 0)
        plsc.subcore_barrier()

        def grp(g, carry):
            base = g * CH
            for k in range(SUB):
                sv = srcb[pl.ds(base + k * 16, 16)]
                dv = dstb[pl.ds(base + k * 16, 16)]
                pa = p0b[pl.ds(base + k * 16, 16)]
                pb_ = p1b[pl.ds(base + k * 16, 16)]
                pc = p2b[pl.ds(base + k * 16, 16)]
                dvb[pl.ds(k * 16, 16)] = dv
                corners = _corners(pa, pb_, pc)
                for c8, (bin_, coef) in enumerate(corners):
                    gis[k][pl.ds(c8 * 16, 16)] = bin_ * Np + sv
                    cf[pl.ds(k * 128 + c8 * 16, 16)] = coef
            copies = [pltpu.async_copy(y_hbm.at[gis[k]], rowss[k], sem)
                      for k in range(SUB)]
            for cp in copies:
                cp.wait()
            for k in range(SUB):
                for e in range(LANES):
                    accs = [None] * (C // 16)
                    for c8 in range(8):
                        cs = plsc.load_gather(
                            cf, [jnp.full((16,), k * 128 + c8 * 16 + e,
                                          jnp.int32)])
                        for blk in range(C // 16):
                            v = rowss[k][c8 * 16 + e, pl.ds(blk * 16, 16)]
                            accs[blk] = (cs * v if accs[blk] is None
                                         else accs[blk] + cs * v)
                    for blk in range(C // 16):
                        msgb[k * 16 + e, pl.ds(blk * 16, 16)] = accs[blk]
            pltpu.sync_copy(msgb, accum.at[dvb], add=True)
            return carry
        lax.fori_loop(0, nchunks, grp, 0)

        plsc.subcore_barrier()

        def cpout(i, carry):
            pltpu.sync_copy(accum.at[pl.ds(row0 + i * 16, 16)], zb)
            pltpu.sync_copy(zb, out_hbm.at[wid, pl.ds(i * 16, 16)])
            return carry
        lax.fori_loop(0, r // 16, cpout, 0)

    return pl.kernel(
        body, mesh=mesh,
        compiler_params=pltpu.CompilerParams(
            needs_layout_passes=False, use_tc_tiling_on_sc=False),
        out_type=jax.ShapeDtypeStruct((NW, r, C), jnp.float32),
        scratch_types=[
            pltpu.VMEM((Epw,), jnp.int32),
            pltpu.VMEM((Epw,), jnp.int32),
            pltpu.VMEM((Epw,), jnp.float32),
            pltpu.VMEM((Epw,), jnp.float32),
            pltpu.VMEM((Epw,), jnp.float32),
            [pltpu.VMEM((8 * LANES,), jnp.int32) for _ in range(4)],
            pltpu.VMEM((4 * 8 * LANES,), jnp.float32),
            pltpu.VMEM((4 * LANES,), jnp.int32),
            [pltpu.VMEM((8 * LANES, C), jnp.float32) for _ in range(4)],
            pltpu.VMEM((4 * LANES, C), jnp.float32),
            pltpu.VMEM((16, C), jnp.float32),
            pltpu.SemaphoreType.DMA,
            pltpu.VMEM_SHARED((Np, C), jnp.float32),
        ],
    )


def _ymat_tc(Np, Cin, C):
    BM = 256

    def body(x_ref, w_ref, o_ref):
        o_ref[...] = jnp.dot(
            x_ref[...], w_ref[0], preferred_element_type=jnp.float32)[None]

    return pl.pallas_call(
        body,
        grid=(125, Np // BM),
        in_specs=[
            pl.BlockSpec((BM, Cin), lambda b, m: (m, 0)),
            pl.BlockSpec((1, Cin, C), lambda b, m: (b, 0, 0)),
        ],
        out_specs=pl.BlockSpec((1, BM, C), lambda b, m: (b, m, 0)),
        out_shape=jax.ShapeDtypeStruct((125, Np, C), jnp.float32),
    )


def _finish_tc(Np, Cin, C, Wn, narrow):
    def body_narrow(a0_ref, a1_ref, x_ref, r_ref, b_ref, f_ref, deg_ref):
        deg = a0_ref[:, C] + a1_ref[:, C]
        d = jnp.maximum(deg, 1.0)
        acc = a0_ref[:, :C] + a1_ref[:, :C]
        f = acc / d[:, None] + jnp.dot(
            x_ref[...], r_ref[...], preferred_element_type=jnp.float32)
        f = f + b_ref[...]
        f_ref[...] = jnp.maximum(f, 0.0)
        deg_ref[...] = deg[:, None]

    def body_wide(a0_ref, a1_ref, x_ref, r_ref, b_ref, deg_ref, f_ref):
        d = jnp.maximum(deg_ref[:, 0], 1.0)
        acc = a0_ref[:, :C] + a1_ref[:, :C]
        f = acc / d[:, None] + jnp.dot(
            x_ref[...], r_ref[...], preferred_element_type=jnp.float32)
        f = f + b_ref[...]
        f_ref[...] = jnp.maximum(f, 0.0)

    if narrow:
        return pl.pallas_call(
            body_narrow,
            out_shape=(jax.ShapeDtypeStruct((Np, C), jnp.float32),
                       jax.ShapeDtypeStruct((Np, 1), jnp.float32)),
        )
    return pl.pallas_call(
        body_wide,
        out_shape=jax.ShapeDtypeStruct((Np, C), jnp.float32),
    )


def _mean_tc(Np, C):
    def body(f_ref, b_ref, o_ref):
        f = f_ref[...]
        bi = b_ref[...]
        rows = []
        for g in range(G):
            m = (bi == g).astype(jnp.float32)
            s = jnp.sum(m * f, axis=0)
            c = jnp.maximum(jnp.sum(m), 1.0)
            rows.append(s / c)
        o_ref[...] = jnp.stack(rows, axis=0)

    return pl.pallas_call(
        body,
        out_shape=jax.ShapeDtypeStruct((G, C), jnp.float32),
    )


def _fc_kernel(xcat_ref, w_ref, b_ref, out_ref):
    logits = jnp.dot(xcat_ref[...], w_ref[...],
                     preferred_element_type=jnp.float32)
    logits = logits + b_ref[...][None, :]
    mx = jnp.max(logits, axis=1, keepdims=True)
    sh = logits - mx
    lse = jnp.log(jnp.sum(jnp.exp(sh), axis=1, keepdims=True))
    out_ref[...] = sh - lse


def _pool_jnp(cur, w, cluster, n_next):
    n = cur.shape[0]
    m = jax.ops.segment_max(w, cluster, num_segments=n_next)
    is_max = w >= m[cluster]
    cand = jnp.where(is_max, jnp.arange(n), n)
    sel = jnp.clip(jax.ops.segment_min(cand, cluster, num_segments=n_next),
                   0, n - 1)
    return cur[sel]


def kernel(x, edge_index1, pseudo1, batch1, cluster1,
           edge_index2, pseudo2, batch2, cluster2,
           edge_index3, pseudo3, batch3, cluster3,
           edge_index4, pseudo4, batch4, cluster4,
           edge_index5, pseudo5, batch5,
           W1, R1, b1, W12, R12, b12,
           W2, R2, b2, W22, R22, b22,
           W3, R3, b3, W32, R32, b32,
           W4, R4, b4, W42, R42, b42,
           W5, R5, b5, W52, R52, b52,
           fcW, fcb):
    d = dict(locals())
    convs = {
        "1": (W1, R1, b1), "12": (W12, R12, b12),
        "2": (W2, R2, b2), "22": (W22, R22, b22),
        "3": (W3, R3, b3), "32": (W32, R32, b32),
        "4": (W4, R4, b4), "42": (W42, R42, b42),
        "5": (W5, R5, b5), "52": (W52, R52, b52),
    }
    pairs = [("1", "12"), ("2", "22"), ("3", "32"), ("4", "42"), ("5", "52")]
    cur = x[:, 0]
    res = []
    for l in range(5):
        N, E = NS[l], ES[l]
        C1 = 32 if l == 0 else 64
        Np = _ceil_to(N + 1, 256)
        Ep = _ceil_to(E, NW * 64)
        a, c = pairs[l]
        Wa, Ra, ba = convs[a]
        Wc, Rc, bc = convs[c]
        ei = d["edge_index%d" % (l + 1)]
        ps = d["pseudo%d" % (l + 1)]
        src = jnp.pad(ei[0].astype(jnp.int32), (0, Ep - E))
        dst = jnp.pad(ei[1].astype(jnp.int32), (0, Ep - E),
                      constant_values=Np - 1)
        p0 = jnp.pad(ps[:, 0], (0, Ep - E))
        p1 = jnp.pad(ps[:, 1], (0, Ep - E))
        p2 = jnp.pad(ps[:, 2], (0, Ep - E))
        xp = jnp.pad(cur, (0, Np - N))

        t_flat = Wa[:, 0, :].reshape(-1)
        o1 = _narrow_sc(Np, Ep, C1)(src, dst, p0, p1, p2, xp, t_flat)
        o1 = o1.reshape(NCORE, Np, C1 + 16)
        f1, deg = _finish_tc(Np, 1, C1, C1 + 16, True)(
            o1[0], o1[1], xp[:, None], Ra, ba[None, :])

        y = _ymat_tc(Np, C1, 64)(f1, Wc)
        yf = y.reshape(125 * Np, 64)
        o2 = _wide_sc(Np, Ep, 64)(src, dst, p0, p1, p2, yf)
        o2 = o2.reshape(NCORE, Np, 64)
        f2 = _finish_tc(Np, C1, 64, 64, False)(
            o2[0], o2[1], f1, Rc, bc[None, :], deg)

        bp = jnp.pad(d["batch%d" % (l + 1)].astype(jnp.int32), (0, Np - N),
                     constant_values=G)[:, None]
        res.append(_mean_tc(Np, 64)(f2, bp))
        if l < 4:
            cur = _pool_jnp(cur, f2[:N, 0],
                            d["cluster%d" % (l + 1)].astype(jnp.int32),
                            NS[l + 1])
    xcat = jnp.concatenate(res, axis=1)
    out = pl.pallas_call(
        _fc_kernel,
        out_shape=jax.ShapeDtypeStruct((G, 10), jnp.float32),
    )(xcat, fcW, fcb)
    return out


# speedup vs baseline: 1.7402x; 1.7402x over previous
import functools

import jax
import jax.numpy as jnp
from jax import lax
from jax.experimental import pallas as pl
from jax.experimental.pallas import tpu as pltpu
from jax.experimental.pallas import tpu_sc as plsc

NS = [10000, 2500, 640, 160, 40]
ES = [160000, 40000, 10000, 2500, 600]
G = 8
K = 5
NCORE = 2
NSUB = 16
NW = NCORE * NSUB
LANES = 16


def _ceil_to(a, m):
    return -(-a // m) * m


def _corners(p0, p1, p2):
    q0, q1, q2 = p0 * 4.0, p1 * 4.0, p2 * 4.0
    lo0 = jnp.minimum(jnp.maximum(q0.astype(jnp.int32), 0), 3)
    lo1 = jnp.minimum(jnp.maximum(q1.astype(jnp.int32), 0), 3)
    lo2 = jnp.minimum(jnp.maximum(q2.astype(jnp.int32), 0), 3)
    fr0 = q0 - lo0.astype(jnp.float32)
    fr1 = q1 - lo1.astype(jnp.float32)
    fr2 = q2 - lo2.astype(jnp.float32)
    out = []
    for bits in range(8):
        b0, b1, b2 = (bits >> 2) & 1, (bits >> 1) & 1, bits & 1
        bin_ = (lo0 + b0) * 25 + (lo1 + b1) * 5 + (lo2 + b2)
        c0 = fr0 if b0 else 1.0 - fr0
        c1 = fr1 if b1 else 1.0 - fr1
        c2 = fr2 if b2 else 1.0 - fr2
        out.append((bin_, c0 * c1 * c2))
    return out


def _narrow_sc(Np, Ep, C):
    Wn = C + 16
    Epw = Ep // NW
    groups = Epw // LANES
    r = Np // NSUB
    mesh = plsc.VectorSubcoreMesh(core_axis_name="c", subcore_axis_name="s")

    def body(src_hbm, dst_hbm, p0_hbm, p1_hbm, p2_hbm, x_hbm, t_hbm, out_hbm,
             srcb, dstb, p0b, p1b, p2b, xb, tb, binb, cfb, msgb, zb,
             accum):
        cid = lax.axis_index("c")
        sid = lax.axis_index("s")
        wid = cid * NSUB + sid
        row0 = sid * r
        pltpu.sync_copy(src_hbm.at[pl.ds(wid * Epw, Epw)], srcb)
        pltpu.sync_copy(dst_hbm.at[pl.ds(wid * Epw, Epw)], dstb)
        pltpu.sync_copy(p0_hbm.at[pl.ds(wid * Epw, Epw)], p0b)
        pltpu.sync_copy(p1_hbm.at[pl.ds(wid * Epw, Epw)], p1b)
        pltpu.sync_copy(p2_hbm.at[pl.ds(wid * Epw, Epw)], p2b)
        pltpu.sync_copy(x_hbm, xb)
        pltpu.sync_copy(t_hbm, tb)

        zeros16 = jnp.zeros((16,), jnp.float32)
        iota = lax.iota(jnp.int32, 16)
        onehot = jnp.where(iota == 0, 1.0, 0.0).astype(jnp.float32)
        for i in range(LANES):
            for j in range(C // 16):
                msgb[i, pl.ds(j * 16, 16)] = zeros16
            msgb[i, pl.ds(C, 16)] = onehot
            for j in range(Wn // 16):
                zb[i, pl.ds(j * 16, 16)] = zeros16

        def zinit(i, carry):
            pltpu.sync_copy(zb, accum.at[pl.ds(row0 + i * 16, 16)])
            return carry
        lax.fori_loop(0, r // 16, zinit, 0)
        plsc.subcore_barrier()

        def grp(g, carry):
            base = g * LANES
            sv = srcb[pl.ds(base, 16)]
            dv = dstb[pl.ds(base, 16)]
            pa = p0b[pl.ds(base, 16)]
            pb_ = p1b[pl.ds(base, 16)]
            pc = p2b[pl.ds(base, 16)]
            xs = plsc.load_gather(xb, [sv])
            corners = _corners(pa, pb_, pc)
            for c8, (bin_, coef) in enumerate(corners):
                binb[pl.ds(c8 * 16, 16)] = bin_ * C
                cfb[pl.ds(c8 * 16, 16)] = coef * xs
            for e in range(LANES):
                accs = [None] * (C // 16)
                for c8 in range(8):
                    pos = jnp.full((16,), c8 * 16 + e, jnp.int32)
                    bs = plsc.load_gather(binb, [pos])
                    cs = plsc.load_gather(cfb, [pos])
                    for blk in range(C // 16):
                        v = plsc.load_gather(tb, [bs + (blk * 16) + iota])
                        accs[blk] = (cs * v if accs[blk] is None
                                     else accs[blk] + cs * v)
                for blk in range(C // 16):
                    msgb[e, pl.ds(blk * 16, 16)] = accs[blk]
            pltpu.sync_copy(msgb, accum.at[dv], add=True)
            return carry
        lax.fori_loop(0, groups, grp, 0)

        plsc.subcore_barrier()

        def cpout(i, carry):
            pltpu.sync_copy(accum.at[pl.ds(row0 + i * 16, 16)], zb)
            pltpu.sync_copy(zb, out_hbm.at[wid, pl.ds(i * 16, 16)])
            return carry
        lax.fori_loop(0, r // 16, cpout, 0)

    return pl.kernel(
        body, mesh=mesh,
        compiler_params=pltpu.CompilerParams(
            needs_layout_passes=False, use_tc_tiling_on_sc=False),
        out_type=jax.ShapeDtypeStruct((NW, r, Wn), jnp.float32),
        scratch_types=[
            pltpu.VMEM((Epw,), jnp.int32),
            pltpu.VMEM((Epw,), jnp.int32),
            pltpu.VMEM((Epw,), jnp.float32),
            pltpu.VMEM((Epw,), jnp.float32),
            pltpu.VMEM((Epw,), jnp.float32),
            pltpu.VMEM((Np,), jnp.float32),
            pltpu.VMEM((125 * C,), jnp.float32),
            pltpu.VMEM((8 * LANES,), jnp.int32),
            pltpu.VMEM((8 * LANES,), jnp.float32),
            pltpu.VMEM((LANES, Wn), jnp.float32),
            pltpu.VMEM((16, Wn), jnp.float32),
            pltpu.VMEM_SHARED((Np, Wn), jnp.float32),
        ],
    )


def _wide_sc(Np, Ep, C):
    SUB = 4
    CH = SUB * LANES
    CQ = 4 * C
    Epw = Ep // NW
    nchunks = Epw // CH
    r = Np // NSUB
    mesh = plsc.VectorSubcoreMesh(core_axis_name="c", subcore_axis_name="s")

    def body(src_hbm, dst_hbm, p0_hbm, p1_hbm, p2_hbm, y_hbm, out_hbm,
             srcb, dstb, p0b, p1b, p2b, gis, cf, dvb, rowss, msgb, zb, sem,
             accum):
        cid = lax.axis_index("c")
        sid = lax.axis_index("s")
        wid = cid * NSUB + sid
        row0 = sid * r
        pltpu.sync_copy(src_hbm.at[pl.ds(wid * Epw, Epw)], srcb)
        pltpu.sync_copy(dst_hbm.at[pl.ds(wid * Epw, Epw)], dstb)
        pltpu.sync_copy(p0_hbm.at[pl.ds(wid * Epw, Epw)], p0b)
        pltpu.sync_copy(p1_hbm.at[pl.ds(wid * Epw, Epw)], p1b)
        pltpu.sync_copy(p2_hbm.at[pl.ds(wid * Epw, Epw)], p2b)

        zeros16 = jnp.zeros((16,), jnp.float32)
        iota = lax.iota(jnp.int32, 16)
        for i in range(16):
            for j in range(C // 16):
                zb[i, pl.ds(j * 16, 16)] = zeros16

        def zinit(i, carry):
            pltpu.sync_copy(zb, accum.at[pl.ds(row0 + i * 16, 16)])
            return carry
        lax.fori_loop(0, r // 16, zinit, 0)
        plsc.subcore_barrier()

        def grp(g, carry):
            base = g * CH
            for k in range(SUB):
                sv = srcb[pl.ds(base + k * 16, 16)]
                dv = dstb[pl.ds(base + k * 16, 16)]
                q0 = p0b[pl.ds(base + k * 16, 16)] * 4.0
                q1 = p1b[pl.ds(base + k * 16, 16)] * 4.0
                q2 = p2b[pl.ds(base + k * 16, 16)] * 4.0
                dvb[pl.ds(k * 16, 16)] = dv
                lo0 = jnp.minimum(jnp.maximum(q0.astype(jnp.int32), 0), 3)
                lo1 = jnp.minimum(jnp.maximum(q1.astype(jnp.int32), 0), 3)
                lo2 = jnp.minimum(jnp.maximum(q2.astype(jnp.int32), 0), 3)
                fr0 = q0 - lo0.astype(jnp.float32)
                fr1 = q1 - lo1.astype(jnp.float32)
                fr2 = q2 - lo2.astype(jnp.float32)
                rbase = sv * 125 + lo1 * 5 + lo2
                qq = [(1.0 - fr1) * (1.0 - fr2), (1.0 - fr1) * fr2,
                      fr1 * (1.0 - fr2), fr1 * fr2]
                for b0 in range(2):
                    gis[k][pl.ds(b0 * 16, 16)] = rbase + (lo0 + b0) * 25
                    c0 = fr0 if b0 else 1.0 - fr0
                    for j in range(4):
                        cf[pl.ds(k * 128 + (b0 * 4 + j) * 16, 16)] = c0 * qq[j]
            copies = [pltpu.async_copy(y_hbm.at[gis[k]], rowss[k], sem)
                      for k in range(SUB)]
            for cp in copies:
                cp.wait()
            for k in range(SUB):
                for e in range(LANES):
                    cs = [plsc.load_gather(
                        cf, [jnp.full((16,), k * 128 + h * 16 + e, jnp.int32)])
                        for h in range(8)]
                    for blk in range(C // 16):
                        acc = None
                        for b0 in range(2):
                            for j in range(4):
                                v = rowss[k][b0 * 16 + e,
                                             pl.ds(j * C + blk * 16, 16)]
                                cv = cs[b0 * 4 + j] * v
                                acc = cv if acc is None else acc + cv
                        msgb[k * 16 + e, pl.ds(blk * 16, 16)] = acc
            pltpu.sync_copy(msgb, accum.at[dvb], add=True)
            return carry
        lax.fori_loop(0, nchunks, grp, 0)

        plsc.subcore_barrier()

        def cpout(i, carry):
            pltpu.sync_copy(accum.at[pl.ds(row0 + i * 16, 16)], zb)
            pltpu.sync_copy(zb, out_hbm.at[wid, pl.ds(i * 16, 16)])
            return carry
        lax.fori_loop(0, r // 16, cpout, 0)

    return pl.kernel(
        body, mesh=mesh,
        compiler_params=pltpu.CompilerParams(
            needs_layout_passes=False, use_tc_tiling_on_sc=False),
        out_type=jax.ShapeDtypeStruct((NW, r, C), jnp.float32),
        scratch_types=[
            pltpu.VMEM((Epw,), jnp.int32),
            pltpu.VMEM((Epw,), jnp.int32),
            pltpu.VMEM((Epw,), jnp.float32),
            pltpu.VMEM((Epw,), jnp.float32),
            pltpu.VMEM((Epw,), jnp.float32),
            [pltpu.VMEM((2 * LANES,), jnp.int32) for _ in range(4)],
            pltpu.VMEM((4 * 8 * LANES,), jnp.float32),
            pltpu.VMEM((4 * LANES,), jnp.int32),
            [pltpu.VMEM((2 * LANES, CQ), jnp.float32) for _ in range(4)],
            pltpu.VMEM((4 * LANES, C), jnp.float32),
            pltpu.VMEM((16, C), jnp.float32),
            pltpu.SemaphoreType.DMA,
            pltpu.VMEM_SHARED((Np, C), jnp.float32),
        ],
    )


def _ymat_tc(Np, Cin, CQ):
    BM = 128
    NQ = 125 * CQ

    def body(x_ref, w_ref, o_ref):
        o_ref[...] = jnp.dot(
            x_ref[...], w_ref[...], preferred_element_type=jnp.float32)

    return pl.pallas_call(
        body,
        grid=(Np // BM,),
        in_specs=[
            pl.BlockSpec((BM, Cin), lambda m: (m, 0)),
            pl.BlockSpec((Cin, NQ), lambda m: (0, 0)),
        ],
        out_specs=pl.BlockSpec((BM, NQ), lambda m: (m, 0)),
        out_shape=jax.ShapeDtypeStruct((Np, NQ), jnp.float32),
    )


def _finish_tc(Np, Cin, C, Wn, narrow):
    def body_narrow(a0_ref, a1_ref, x_ref, r_ref, b_ref, f_ref, deg_ref):
        deg = a0_ref[:, C] + a1_ref[:, C]
        d = jnp.maximum(deg, 1.0)
        acc = a0_ref[:, :C] + a1_ref[:, :C]
        f = acc / d[:, None] + jnp.dot(
            x_ref[...], r_ref[...], preferred_element_type=jnp.float32)
        f = f + b_ref[...]
        f_ref[...] = jnp.maximum(f, 0.0)
        deg_ref[...] = deg[:, None]

    def body_wide(a0_ref, a1_ref, x_ref, r_ref, b_ref, deg_ref, f_ref):
        d = jnp.maximum(deg_ref[:, 0], 1.0)
        acc = a0_ref[:, :C] + a1_ref[:, :C]
        f = acc / d[:, None] + jnp.dot(
            x_ref[...], r_ref[...], preferred_element_type=jnp.float32)
        f = f + b_ref[...]
        f_ref[...] = jnp.maximum(f, 0.0)

    if narrow:
        return pl.pallas_call(
            body_narrow,
            out_shape=(jax.ShapeDtypeStruct((Np, C), jnp.float32),
                       jax.ShapeDtypeStruct((Np, 1), jnp.float32)),
        )
    return pl.pallas_call(
        body_wide,
        out_shape=jax.ShapeDtypeStruct((Np, C), jnp.float32),
    )


def _mean_tc(Np, C):
    def body(f_ref, b_ref, o_ref):
        f = f_ref[...]
        bi = b_ref[...]
        rows = []
        for g in range(G):
            m = (bi == g).astype(jnp.float32)
            s = jnp.sum(m * f, axis=0)
            c = jnp.maximum(jnp.sum(m), 1.0)
            rows.append(s / c)
        o_ref[...] = jnp.stack(rows, axis=0)

    return pl.pallas_call(
        body,
        out_shape=jax.ShapeDtypeStruct((G, C), jnp.float32),
    )


def _fc_kernel(xcat_ref, w_ref, b_ref, out_ref):
    logits = jnp.dot(xcat_ref[...], w_ref[...],
                     preferred_element_type=jnp.float32)
    logits = logits + b_ref[...][None, :]
    mx = jnp.max(logits, axis=1, keepdims=True)
    sh = logits - mx
    lse = jnp.log(jnp.sum(jnp.exp(sh), axis=1, keepdims=True))
    out_ref[...] = sh - lse


def _pool_jnp(cur, w, cluster, n_next):
    n = cur.shape[0]
    m = jax.ops.segment_max(w, cluster, num_segments=n_next)
    is_max = w >= m[cluster]
    cand = jnp.where(is_max, jnp.arange(n), n)
    sel = jnp.clip(jax.ops.segment_min(cand, cluster, num_segments=n_next),
                   0, n - 1)
    return cur[sel]


def kernel(x, edge_index1, pseudo1, batch1, cluster1,
           edge_index2, pseudo2, batch2, cluster2,
           edge_index3, pseudo3, batch3, cluster3,
           edge_index4, pseudo4, batch4, cluster4,
           edge_index5, pseudo5, batch5,
           W1, R1, b1, W12, R12, b12,
           W2, R2, b2, W22, R22, b22,
           W3, R3, b3, W32, R32, b32,
           W4, R4, b4, W42, R42, b42,
           W5, R5, b5, W52, R52, b52,
           fcW, fcb):
    d = dict(locals())
    convs = {
        "1": (W1, R1, b1), "12": (W12, R12, b12),
        "2": (W2, R2, b2), "22": (W22, R22, b22),
        "3": (W3, R3, b3), "32": (W32, R32, b32),
        "4": (W4, R4, b4), "42": (W42, R42, b42),
        "5": (W5, R5, b5), "52": (W52, R52, b52),
    }
    pairs = [("1", "12"), ("2", "22"), ("3", "32"), ("4", "42"), ("5", "52")]
    cur = x[:, 0]
    res = []
    for l in range(5):
        N, E = NS[l], ES[l]
        C1 = 32 if l == 0 else 64
        Np = _ceil_to(N + 1, 256)
        Ep = _ceil_to(E, NW * 64)
        a, c = pairs[l]
        Wa, Ra, ba = convs[a]
        Wc, Rc, bc = convs[c]
        ei = d["edge_index%d" % (l + 1)]
        ps = d["pseudo%d" % (l + 1)]
        src = jnp.pad(ei[0].astype(jnp.int32), (0, Ep - E))
        dst = jnp.pad(ei[1].astype(jnp.int32), (0, Ep - E),
                      constant_values=Np - 1)
        p0 = jnp.pad(ps[:, 0], (0, Ep - E))
        p1 = jnp.pad(ps[:, 1], (0, Ep - E))
        p2 = jnp.pad(ps[:, 2], (0, Ep - E))
        xp = jnp.pad(cur, (0, Np - N))

        t_flat = Wa[:, 0, :].reshape(-1)
        o1 = _narrow_sc(Np, Ep, C1)(src, dst, p0, p1, p2, xp, t_flat)
        o1 = o1.reshape(NCORE, Np, C1 + 16)
        f1, deg = _finish_tc(Np, 1, C1, C1 + 16, True)(
            o1[0], o1[1], xp[:, None], Ra, ba[None, :])

        Wq = jnp.concatenate(
            [Wc, jnp.roll(Wc, -1, 0), jnp.roll(Wc, -5, 0),
             jnp.roll(Wc, -6, 0)], axis=2)
        Wq2 = Wq.transpose(1, 0, 2).reshape(C1, 125 * 256)
        y = _ymat_tc(Np, C1, 256)(f1, Wq2)
        yf = y.reshape(Np * 125, 256)
        o2 = _wide_sc(Np, Ep, 64)(src, dst, p0, p1, p2, yf)
        o2 = o2.reshape(NCORE, Np, 64)
        f2 = _finish_tc(Np, C1, 64, 64, False)(
            o2[0], o2[1], f1, Rc, bc[None, :], deg)

        bp = jnp.pad(d["batch%d" % (l + 1)].astype(jnp.int32), (0, Np - N),
                     constant_values=G)[:, None]
        res.append(_mean_tc(Np, 64)(f2, bp))
        if l < 4:
            cur = _pool_jnp(cur, f2[:N, 0],
                            d["cluster%d" % (l + 1)].astype(jnp.int32),
                            NS[l + 1])
    xcat = jnp.concatenate(res, axis=1)
    out = pl.pallas_call(
        _fc_kernel,
        out_shape=jax.ShapeDtypeStruct((G, 10), jnp.float32),
    )(xcat, fcW, fcb)
    return out
